# baseline (speedup 1.0000x reference)
import functools
import math

import jax
import jax.numpy as jnp
from jax import lax
from jax.experimental import pallas as pl
from jax.experimental.pallas import tpu as pltpu
from jax.experimental.pallas import tpu_sc as plsc

D_MODEL = 64
SCALE = math.sqrt(D_MODEL)
LANES = 16

_NC = 2
_NS = 16
_NW = _NC * _NS


def _make_gather_kernel(B, CH):
    assert B % (_NW * CH) == 0
    b_per_w = B // _NW
    nchunks = b_per_w // CH

    mesh = plsc.VectorSubcoreMesh(core_axis_name="c", subcore_axis_name="s")

    @functools.partial(
        pl.kernel,
        mesh=mesh,
        out_type=jax.ShapeDtypeStruct((B, D_MODEL), jnp.float32),
        compiler_params=pltpu.CompilerParams(use_tc_tiling_on_sc=False),
        scratch_types=[
            pltpu.VMEM((b_per_w,), jnp.int32),
            pltpu.VMEM((CH, D_MODEL), jnp.float32),
            pltpu.SemaphoreType.DMA,
        ],
    )
    def gather_scale(idx_hbm, table_hbm, out_hbm, idx_v, rows, sem):
        wid = lax.axis_index("s") * _NC + lax.axis_index("c")
        base = wid * b_per_w
        pltpu.sync_copy(idx_hbm.at[pl.ds(base, b_per_w)], idx_v)

        def chunk_body(c, carry):
            pltpu.async_copy(
                table_hbm.at[idx_v.at[pl.ds(c * CH, CH)]], rows, sem
            ).wait()

            def row_body(i, rcarry):
                for j in range(D_MODEL // LANES):
                    sl = pl.ds(j * LANES, LANES)
                    rows[i, sl] = rows[i, sl] * SCALE
                return rcarry

            lax.fori_loop(0, CH, row_body, 0, unroll=4)

            pltpu.sync_copy(rows, out_hbm.at[pl.ds(base + c * CH, CH)])
            return carry

        lax.fori_loop(0, nchunks, chunk_body, 0)

    return gather_scale


def kernel(x, table):
    S0, S1 = x.shape
    B = S0 * S1
    idx = x.reshape(B).astype(jnp.int32)
    gather_scale = _make_gather_kernel(B, CH=512)
    out = gather_scale(idx, table)
    return out.reshape(S0, S1, D_MODEL)


# speedup vs baseline: 1.0709x; 1.0709x over previous
import functools
import math

import jax
import jax.numpy as jnp
from jax import lax
from jax.experimental import pallas as pl
from jax.experimental.pallas import tpu as pltpu
from jax.experimental.pallas import tpu_sc as plsc

D_MODEL = 64
SCALE = math.sqrt(D_MODEL)
LANES = 16

_NC = 2
_NS = 16
_NW = _NC * _NS
_NBUF = 4


def _make_gather_kernel(B, CH):
    assert B % (_NW * CH) == 0
    b_per_w = B // _NW
    nchunks = b_per_w // CH
    assert nchunks % _NBUF == 0 and nchunks >= 2 * _NBUF

    mesh = plsc.VectorSubcoreMesh(core_axis_name="c", subcore_axis_name="s")

    @functools.partial(
        pl.kernel,
        mesh=mesh,
        out_type=jax.ShapeDtypeStruct((B, D_MODEL), jnp.float32),
        compiler_params=pltpu.CompilerParams(use_tc_tiling_on_sc=False),
        scratch_types=[
            pltpu.VMEM((b_per_w,), jnp.int32),
        ]
        + [pltpu.VMEM((CH, D_MODEL), jnp.float32) for _ in range(_NBUF)]
        + [pltpu.SemaphoreType.DMA for _ in range(2 * _NBUF)],
    )
    def gather_scale(idx_hbm, table_hbm, out_hbm, idx_v, *bufs_and_sems):
        rows = bufs_and_sems[:_NBUF]
        gsem = bufs_and_sems[_NBUF : 2 * _NBUF]
        ssem = bufs_and_sems[2 * _NBUF :]

        wid = lax.axis_index("s") * _NC + lax.axis_index("c")
        base = wid * b_per_w
        pltpu.sync_copy(idx_hbm.at[pl.ds(base, b_per_w)], idx_v)

        def gather(c, b):
            return pltpu.make_async_copy(
                table_hbm.at[idx_v.at[pl.ds(c * CH, CH)]], rows[b], gsem[b]
            )

        def store(c, b):
            return pltpu.make_async_copy(
                rows[b], out_hbm.at[pl.ds(base + c * CH, CH)], ssem[b]
            )

        def scale(b):
            def row_body(i, carry):
                for j in range(D_MODEL // LANES):
                    sl = pl.ds(j * LANES, LANES)
                    rows[b][i, sl] = rows[b][i, sl] * SCALE
                return carry

            lax.fori_loop(0, CH, row_body, 0, unroll=8)

        DIST = _NBUF - 2

        for b in range(DIST):
            gather(b, b).start()

        def outer(i, carry):
            c0 = i * _NBUF
            for b in range(_NBUF):
                c = c0 + b
                g = c + DIST
                gb = (b + DIST) % _NBUF

                @pl.when(jnp.logical_and(g < nchunks, g >= _NBUF))
                def _():
                    store(g - _NBUF, gb).wait()

                @pl.when(g < nchunks)
                def _():
                    gather(g, gb).start()

                gather(c, b).wait()
                scale(b)
                store(c, b).start()
            return carry

        lax.fori_loop(0, nchunks // _NBUF, outer, 0)

        for b in range(_NBUF):
            store(nchunks - _NBUF + b, b).wait()

    return gather_scale


def kernel(x, table):
    S0, S1 = x.shape
    B = S0 * S1
    idx = x.reshape(B).astype(jnp.int32)
    gather_scale = _make_gather_kernel(B, CH=256)
    out = gather_scale(idx, table)
    return out.reshape(S0, S1, D_MODEL)
